# Initial kernel scaffold; baseline (speedup 1.0000x reference)
import jax
import jax.numpy as jnp
from jax.experimental import pallas as pl


def kernel(indices, table):
    raise NotImplementedError("write your pallas kernel here")


# speedup vs baseline: 9.7591x; 9.7591x over previous
import functools

import jax
import jax.numpy as jnp
from jax import lax
from jax.experimental import pallas as pl
from jax.experimental.pallas import tpu as pltpu
from jax.experimental.pallas import tpu_sc as plsc

B = 4096
LK = 512
D = 128
V = 2048
NC, NS = 2, 16
NW = NC * NS
BPW = B // NW
NLANE = 16
DCH = D // NLANE

_mesh = plsc.VectorSubcoreMesh(core_axis_name="c", subcore_axis_name="s")


@functools.partial(
    pl.kernel,
    out_type=jax.ShapeDtypeStruct((B, D), jnp.float32),
    mesh=_mesh,
    scratch_types=[
        pltpu.VMEM((LK,), jnp.int32),
        pltpu.VMEM((LK, D), jnp.float32),
        pltpu.VMEM((D,), jnp.float32),
        pltpu.SemaphoreType.DMA,
    ],
)
def _gather_mean(idx_hbm, table_hbm, out_hbm, idx_v, rows_v, orow_v, sem):
    wid = lax.axis_index("s") * NC + lax.axis_index("c")
    base = wid * BPW

    def row_body(r, carry):
        row = base + r
        pltpu.sync_copy(idx_hbm.at[row], idx_v)
        pltpu.async_copy(table_hbm.at[idx_v], rows_v, sem).wait()

        def add_body(l, accs):
            return tuple(
                acc + rows_v[l, pl.ds(c * NLANE, NLANE)]
                for c, acc in enumerate(accs)
            )

        accs = lax.fori_loop(
            0, LK, add_body,
            tuple(jnp.zeros((NLANE,), jnp.float32) for _ in range(DCH)),
        )
        for c in range(DCH):
            orow_v[pl.ds(c * NLANE, NLANE)] = accs[c] * (1.0 / LK)
        pltpu.sync_copy(orow_v, out_hbm.at[row])
        return carry

    lax.fori_loop(0, BPW, row_body, 0)


def kernel(indices, table):
    return _gather_mean(indices.astype(jnp.int32), table)


# speedup vs baseline: 46.9830x; 4.8143x over previous
import functools

import jax
import jax.numpy as jnp
from jax import lax
from jax.experimental import pallas as pl
from jax.experimental.pallas import tpu as pltpu
from jax.experimental.pallas import tpu_sc as plsc

B = 4096
LK = 512
D = 128
V = 2048
NC, NS = 2, 16
NW = NC * NS
BPW = B // NW
NLANE = 16
GRP = BPW // NLANE
UNROLL = 8

_mesh = plsc.VectorSubcoreMesh(core_axis_name="c", subcore_axis_name="s")


@functools.partial(
    pl.kernel,
    out_type=jax.ShapeDtypeStruct((B * V,), jnp.float32),
    mesh=_mesh,
    scratch_types=[
        pltpu.VMEM((NLANE * LK,), jnp.int32),
        pltpu.VMEM((NLANE * V,), jnp.float32),
    ],
    compiler_params=pltpu.CompilerParams(needs_layout_passes=False),
)
def _hist(idx_hbm, h_hbm, idxg_v, hist_v):
    wid = lax.axis_index("s") * NC + lax.axis_index("c")
    base = wid * BPW
    lane = lax.iota(jnp.int32, NLANE)
    lane_lk = lane * LK
    lane_v = lane * V
    ones = jnp.ones((NLANE,), jnp.float32)
    zeros = jnp.zeros((NLANE,), jnp.float32)

    def group_body(g, carry):
        gbase = base + g * NLANE
        pltpu.sync_copy(idx_hbm.at[pl.ds(gbase * LK, NLANE * LK)], idxg_v)

        def zero_body(c, cc):
            for j in range(UNROLL):
                hist_v[pl.ds((c * UNROLL + j) * NLANE, NLANE)] = zeros
            return cc

        lax.fori_loop(0, NLANE * V // (NLANE * UNROLL), zero_body, 0)

        def scat_body(i, cc):
            for k in range(UNROLL):
                l = i * UNROLL + k
                ids = plsc.load_gather(idxg_v, [lane_lk + l])
                plsc.addupdate_scatter(hist_v, [lane_v + ids], ones)
            return cc

        lax.fori_loop(0, LK // UNROLL, scat_body, 0)
        pltpu.sync_copy(hist_v, h_hbm.at[pl.ds(gbase * V, NLANE * V)])
        return carry

    lax.fori_loop(0, GRP, group_body, 0)


def _matmul_body(h_ref, t_ref, o_ref):
    o_ref[...] = jnp.dot(
        h_ref[...], t_ref[...], preferred_element_type=jnp.float32,
    ) * (1.0 / LK)


_BM = 512

_matmul = pl.pallas_call(
    _matmul_body,
    grid=(B // _BM,),
    in_specs=[
        pl.BlockSpec((_BM, V), lambda i: (i, 0)),
        pl.BlockSpec((V, D), lambda i: (0, 0)),
    ],
    out_specs=pl.BlockSpec((_BM, D), lambda i: (i, 0)),
    out_shape=jax.ShapeDtypeStruct((B, D), jnp.float32),
    compiler_params=pltpu.CompilerParams(
        dimension_semantics=("arbitrary",),
    ),
)


def kernel(indices, table):
    h = _hist(indices.astype(jnp.int32).reshape(-1))
    return _matmul(h.reshape(B, V), table)


# speedup vs baseline: 61.4061x; 1.3070x over previous
import functools

import jax
import jax.numpy as jnp
from jax import lax
from jax.experimental import pallas as pl
from jax.experimental.pallas import tpu as pltpu
from jax.experimental.pallas import tpu_sc as plsc

B = 4096
LK = 512
D = 128
V = 2048
VH = V // 2
NC, NS = 2, 16
NW = NC * NS
BPW = B // NW
NLANE = 16
GRP = BPW // NLANE
UNROLL = 8

_mesh = plsc.VectorSubcoreMesh(core_axis_name="c", subcore_axis_name="s")


@functools.partial(
    pl.kernel,
    out_type=jax.ShapeDtypeStruct((B * VH,), jnp.int32),
    mesh=_mesh,
    scratch_types=[
        pltpu.VMEM((NLANE * LK,), jnp.int32),
        pltpu.VMEM((NLANE * LK,), jnp.int32),
        pltpu.VMEM((NLANE * VH,), jnp.int32),
        pltpu.VMEM((NLANE * VH,), jnp.int32),
        pltpu.SemaphoreType.DMA,
        pltpu.SemaphoreType.DMA,
        pltpu.SemaphoreType.DMA,
        pltpu.SemaphoreType.DMA,
    ],
    compiler_params=pltpu.CompilerParams(needs_layout_passes=False),
)
def _hist(idx_hbm, h_hbm, idx0, idx1, h0, h1, si0, si1, so0, so1):
    wid = lax.axis_index("s") * NC + lax.axis_index("c")
    base = wid * BPW
    lane = lax.iota(jnp.int32, NLANE)
    lane_lk = lane * LK
    lane_vh = lane * VH
    zeros = jnp.zeros((NLANE,), jnp.int32)
    one = jnp.int32(1)

    idxb = [idx0, idx1]
    hb = [h0, h1]
    sis = [si0, si1]
    sos = [so0, so1]
    idx_desc = [None, None]
    out_desc = [None, None]

    idx_desc[0] = pltpu.async_copy(
        idx_hbm.at[pl.ds(base * LK, NLANE * LK)], idx0, si0)

    for g in range(GRP):
        p = g % 2
        gbase = base + g * NLANE
        if g + 1 < GRP:
            nbase = base + (g + 1) * NLANE
            idx_desc[1 - p] = pltpu.async_copy(
                idx_hbm.at[pl.ds(nbase * LK, NLANE * LK)],
                idxb[1 - p], sis[1 - p])
        idx_desc[p].wait()
        if out_desc[p] is not None:
            out_desc[p].wait()
        hv = hb[p]
        iv = idxb[p]

        def zero_body(c, cc, hv=hv):
            for j in range(UNROLL):
                hv[pl.ds((c * UNROLL + j) * NLANE, NLANE)] = zeros
            return cc

        lax.fori_loop(0, VH // UNROLL, zero_body, 0)

        def scat_body(i, cc, hv=hv, iv=iv):
            for k in range(UNROLL):
                l = i * UNROLL + k
                ids = plsc.load_gather(iv, [lane_lk + l])
                val = one << ((ids >> 10) << 4)
                addr = lane_vh + (ids & (VH - 1))
                plsc.addupdate_scatter(hv, [addr], val)
            return cc

        lax.fori_loop(0, LK // UNROLL, scat_body, 0)
        out_desc[p] = pltpu.async_copy(
            hv, h_hbm.at[pl.ds(gbase * VH, NLANE * VH)], sos[p])

    out_desc[0].wait()
    out_desc[1].wait()


def _matmul_body(h_ref, t_ref, o_ref):
    h = h_ref[...]
    hlo = (h & 0xFFFF).astype(jnp.float32)
    hhi = lax.shift_right_logical(h, 16).astype(jnp.float32)
    acc = jnp.dot(hlo, t_ref[:VH, :], preferred_element_type=jnp.float32)
    acc = acc + jnp.dot(hhi, t_ref[VH:, :],
                        preferred_element_type=jnp.float32)
    o_ref[...] = acc * (1.0 / LK)


_BM = 512

_matmul = pl.pallas_call(
    _matmul_body,
    grid=(B // _BM,),
    in_specs=[
        pl.BlockSpec((_BM, VH), lambda i: (i, 0)),
        pl.BlockSpec((V, D), lambda i: (0, 0)),
    ],
    out_specs=pl.BlockSpec((_BM, D), lambda i: (i, 0)),
    out_shape=jax.ShapeDtypeStruct((B, D), jnp.float32),
    compiler_params=pltpu.CompilerParams(
        dimension_semantics=("arbitrary",),
    ),
)


def kernel(indices, table):
    h = _hist(indices.astype(jnp.int32).reshape(-1))
    return _matmul(h.reshape(B, VH), table)


# speedup vs baseline: 80.7113x; 1.3144x over previous
import functools

import jax
import jax.numpy as jnp
from jax import lax
from jax.experimental import pallas as pl
from jax.experimental.pallas import tpu as pltpu
from jax.experimental.pallas import tpu_sc as plsc

B = 4096
LK = 512
D = 128
V = 2048
VH = V // 2
NC, NS = 2, 16
NW = NC * NS
BPW = B // NW
NLANE = 16
GRP = BPW // NLANE
UNROLL = 8

_mesh = plsc.VectorSubcoreMesh(core_axis_name="c", subcore_axis_name="s")


@functools.partial(
    pl.kernel,
    out_type=jax.ShapeDtypeStruct((B * VH,), jnp.int32),
    mesh=_mesh,
    scratch_types=[
        pltpu.VMEM((NLANE * LK,), jnp.int32),
        pltpu.VMEM((NLANE * LK,), jnp.int32),
        pltpu.VMEM((NLANE * VH,), jnp.int32),
        pltpu.VMEM((NLANE * VH,), jnp.int32),
        pltpu.SemaphoreType.DMA,
        pltpu.SemaphoreType.DMA,
        pltpu.SemaphoreType.DMA,
        pltpu.SemaphoreType.DMA,
    ],
    compiler_params=pltpu.CompilerParams(needs_layout_passes=False),
)
def _hist(idx_hbm, h_hbm, idx0, idx1, h0, h1, si0, si1, so0, so1):
    wid = lax.axis_index("s") * NC + lax.axis_index("c")
    base = wid * BPW
    lane = lax.iota(jnp.int32, NLANE)
    lane_lk = lane * LK
    lane_vh = lane * VH
    zeros = jnp.zeros((NLANE,), jnp.int32)
    one = jnp.int32(1)

    idxb = [idx0, idx1]
    hb = [h0, h1]
    sis = [si0, si1]
    sos = [so0, so1]
    idx_desc = [None, None]
    out_desc = [None, None]

    idx_desc[0] = pltpu.async_copy(
        idx_hbm.at[pl.ds(base * LK, NLANE * LK)], idx0, si0)

    for g in range(GRP):
        p = g % 2
        gbase = base + g * NLANE
        if g + 1 < GRP:
            nbase = base + (g + 1) * NLANE
            idx_desc[1 - p] = pltpu.async_copy(
                idx_hbm.at[pl.ds(nbase * LK, NLANE * LK)],
                idxb[1 - p], sis[1 - p])
        idx_desc[p].wait()
        if out_desc[p] is not None:
            out_desc[p].wait()
        hv = hb[p]
        iv = idxb[p]

        def zero_body(c, cc, hv=hv):
            for j in range(UNROLL):
                hv[pl.ds((c * UNROLL + j) * NLANE, NLANE)] = zeros
            return cc

        lax.fori_loop(0, VH // UNROLL, zero_body, 0)

        def scat_body(i, cc, hv=hv, iv=iv):
            for k in range(UNROLL):
                l = i * UNROLL + k
                tok = (l + lane) & (LK - 1)
                ids = plsc.load_gather(iv, [lane_lk + tok])
                val = one + (ids >> 10) * jnp.int32(0xFFFF)
                addr = lane_vh + (ids & (VH - 1))
                plsc.addupdate_scatter(hv, [addr], val)
            return cc

        lax.fori_loop(0, LK // UNROLL, scat_body, 0)
        out_desc[p] = pltpu.async_copy(
            hv, h_hbm.at[pl.ds(gbase * VH, NLANE * VH)], sos[p])

    out_desc[0].wait()
    out_desc[1].wait()


def _matmul_body(h_ref, t_ref, o_ref):
    h = h_ref[...]
    hlo = (h & 0xFFFF).astype(jnp.float32)
    hhi = lax.shift_right_logical(h, 16).astype(jnp.float32)
    acc = jnp.dot(hlo, t_ref[:VH, :], preferred_element_type=jnp.float32)
    acc = acc + jnp.dot(hhi, t_ref[VH:, :],
                        preferred_element_type=jnp.float32)
    o_ref[...] = acc * (1.0 / LK)


_BM = 512

_matmul = pl.pallas_call(
    _matmul_body,
    grid=(B // _BM,),
    in_specs=[
        pl.BlockSpec((_BM, VH), lambda i: (i, 0)),
        pl.BlockSpec((V, D), lambda i: (0, 0)),
    ],
    out_specs=pl.BlockSpec((_BM, D), lambda i: (i, 0)),
    out_shape=jax.ShapeDtypeStruct((B, D), jnp.float32),
    compiler_params=pltpu.CompilerParams(
        dimension_semantics=("arbitrary",),
    ),
)


def kernel(indices, table):
    h = _hist(indices.astype(jnp.int32).reshape(-1))
    return _matmul(h.reshape(B, VH), table)


# speedup vs baseline: 108.7394x; 1.3473x over previous
import functools

import jax
import jax.numpy as jnp
from jax import lax
from jax.experimental import pallas as pl
from jax.experimental.pallas import tpu as pltpu
from jax.experimental.pallas import tpu_sc as plsc

B = 4096
LK = 512
D = 128
V = 2048
VH = V // 2
NC, NS = 2, 16
NW = NC * NS
BPW = B // NW
NLANE = 16
GRP = BPW // NLANE
UNROLL = 8

_mesh = plsc.VectorSubcoreMesh(core_axis_name="c", subcore_axis_name="s")


@functools.partial(
    pl.kernel,
    out_type=jax.ShapeDtypeStruct((B * VH,), jnp.int32),
    mesh=_mesh,
    scratch_types=[
        pltpu.VMEM((NLANE * LK,), jnp.int32),
        pltpu.VMEM((NLANE * LK,), jnp.int32),
        pltpu.VMEM((NLANE * VH,), jnp.int32),
        pltpu.VMEM((NLANE * VH,), jnp.int32),
        pltpu.SemaphoreType.DMA,
        pltpu.SemaphoreType.DMA,
        pltpu.SemaphoreType.DMA,
        pltpu.SemaphoreType.DMA,
    ],
    compiler_params=pltpu.CompilerParams(needs_layout_passes=False),
)
def _hist(idx_hbm, h_hbm, idx0, idx1, h0, h1, si0, si1, so0, so1):
    wid = lax.axis_index("s") * NC + lax.axis_index("c")
    base = wid * BPW
    lane = lax.iota(jnp.int32, NLANE)
    lane_lk = lane * LK
    lane_vh = lane * VH
    zeros = jnp.zeros((NLANE,), jnp.int32)
    one = jnp.int32(1)

    idxb = [idx0, idx1]
    hb = [h0, h1]
    sis = [si0, si1]
    sos = [so0, so1]
    idx_desc = [None, None]
    out_desc = [None, None]

    idx_desc[0] = pltpu.async_copy(
        idx_hbm.at[pl.ds(base * LK, NLANE * LK)], idx0, si0)

    for g in range(GRP):
        p = g % 2
        gbase = base + g * NLANE
        if g + 1 < GRP:
            nbase = base + (g + 1) * NLANE
            idx_desc[1 - p] = pltpu.async_copy(
                idx_hbm.at[pl.ds(nbase * LK, NLANE * LK)],
                idxb[1 - p], sis[1 - p])
        idx_desc[p].wait()
        if out_desc[p] is not None:
            out_desc[p].wait()
        hv = hb[p]
        iv = idxb[p]

        @plsc.parallel_loop(0, NLANE * VH, step=NLANE, unroll=UNROLL)
        def zero_body(c, hv=hv):
            hv[pl.ds(c, NLANE)] = zeros

        @plsc.parallel_loop(0, LK, step=1, unroll=UNROLL)
        def scat_body(l, hv=hv, iv=iv):
            tok = (l + lane) & (LK - 1)
            ids = plsc.load_gather(iv, [lane_lk + tok])
            val = one + (ids >> 10) * jnp.int32(0xFFFF)
            addr = lane_vh + (ids & (VH - 1))
            plsc.addupdate_scatter(hv, [addr], val)
        out_desc[p] = pltpu.async_copy(
            hv, h_hbm.at[pl.ds(gbase * VH, NLANE * VH)], sos[p])

    out_desc[0].wait()
    out_desc[1].wait()


def _matmul_body(h_ref, t_ref, o_ref):
    h = h_ref[...]
    hlo = (h & 0xFFFF).astype(jnp.float32)
    hhi = lax.shift_right_logical(h, 16).astype(jnp.float32)
    acc = jnp.dot(hlo, t_ref[:VH, :], preferred_element_type=jnp.float32)
    acc = acc + jnp.dot(hhi, t_ref[VH:, :],
                        preferred_element_type=jnp.float32)
    o_ref[...] = acc * (1.0 / LK)


_BM = 512

_matmul = pl.pallas_call(
    _matmul_body,
    grid=(B // _BM,),
    in_specs=[
        pl.BlockSpec((_BM, VH), lambda i: (i, 0)),
        pl.BlockSpec((V, D), lambda i: (0, 0)),
    ],
    out_specs=pl.BlockSpec((_BM, D), lambda i: (i, 0)),
    out_shape=jax.ShapeDtypeStruct((B, D), jnp.float32),
    compiler_params=pltpu.CompilerParams(
        dimension_semantics=("arbitrary",),
    ),
)


def kernel(indices, table):
    h = _hist(indices.astype(jnp.int32).reshape(-1))
    return _matmul(h.reshape(B, VH), table)
